# baseline (speedup 1.0000x reference)
import jax
import jax.numpy as jnp
from jax.experimental import pallas as pl

_EMBED = 4096
_NE = 64
_K = 8
_NT = 32768
_BT = 512


def _body(x_ref, whl_ref, b_ref, p_ref, idx_ref):
    x = x_ref[...]
    xh = x.astype(jnp.bfloat16)
    acc = jnp.dot(xh, whl_ref[...], preferred_element_type=jnp.float32)
    logits = acc + b_ref[...]

    lt = logits.T
    m = jnp.max(lt, axis=0, keepdims=True)
    e = jnp.exp(lt - m)
    s = jnp.sum(e, axis=0, keepdims=True)
    p_ref[...] = (e / s).T

    vals = lt
    iota = jax.lax.broadcasted_iota(jnp.int32, (_NE, lt.shape[1]), 0)
    rows = []
    for _ in range(_K):
        mx = jnp.max(vals, axis=0, keepdims=True)
        amin = jnp.min(jnp.where(vals >= mx, iota, _NE), axis=0, keepdims=True)
        rows.append(amin)
        vals = jnp.where(iota == amin, -jnp.inf, vals)
    idx_ref[...] = jnp.concatenate(rows, axis=0).T


@jax.jit
def kernel(inputs, W, b):
    wt = W.T
    whl = wt.astype(jnp.bfloat16)
    bb = b.reshape(1, _NE)
    grid = (_NT // _BT,)
    p, idx = pl.pallas_call(
        _body,
        grid=grid,
        in_specs=[
            pl.BlockSpec((_BT, _EMBED), lambda i: (i, 0)),
            pl.BlockSpec((_EMBED, _NE), lambda i: (0, 0)),
            pl.BlockSpec((1, _NE), lambda i: (0, 0)),
        ],
        out_specs=[
            pl.BlockSpec((_BT, _NE), lambda i: (i, 0)),
            pl.BlockSpec((_BT, _K), lambda i: (i, 0)),
        ],
        out_shape=[
            jax.ShapeDtypeStruct((_NT, _NE), jnp.float32),
            jax.ShapeDtypeStruct((_NT, _K), jnp.int32),
        ],
    )(inputs, whl, bb)
    return (p, idx)


# speedup vs baseline: 1.0552x; 1.0552x over previous
import jax
import jax.numpy as jnp
from jax.experimental import pallas as pl

_EMBED = 4096
_NE = 64
_K = 8
_NT = 32768
_BT = 1024


def _body(x1_ref, x2_ref, whl_ref, b_ref, p_ref, idx_ref):
    xh1 = x1_ref[...].astype(jnp.bfloat16)
    xh2 = x2_ref[...].astype(jnp.bfloat16)
    w = whl_ref[...]
    acc = jnp.dot(xh1, w[: _EMBED // 2], preferred_element_type=jnp.float32)
    acc += jnp.dot(xh2, w[_EMBED // 2 :], preferred_element_type=jnp.float32)
    logits = acc + b_ref[...]

    lt = logits.T
    m = jnp.max(lt, axis=0, keepdims=True)
    e = jnp.exp(lt - m)
    s = jnp.sum(e, axis=0, keepdims=True)
    p_ref[...] = (e / s).T

    vals = lt
    iota = jax.lax.broadcasted_iota(jnp.int32, (_NE, lt.shape[1]), 0)
    rows = []
    for _ in range(_K):
        mx = jnp.max(vals, axis=0, keepdims=True)
        amin = jnp.min(jnp.where(vals >= mx, iota, _NE), axis=0, keepdims=True)
        rows.append(amin)
        vals = jnp.where(iota == amin, -jnp.inf, vals)
    idx_ref[...] = jnp.concatenate(rows, axis=0).T


@jax.jit
def kernel(inputs, W, b):
    wt = W.T
    whl = wt.astype(jnp.bfloat16)
    bb = b.reshape(1, _NE)
    grid = (_NT // _BT,)
    p, idx = pl.pallas_call(
        _body,
        grid=grid,
        in_specs=[
            pl.BlockSpec((_BT, _EMBED // 2), lambda i: (i, 0)),
            pl.BlockSpec((_BT, _EMBED // 2), lambda i: (i, 1)),
            pl.BlockSpec((_EMBED, _NE), lambda i: (0, 0)),
            pl.BlockSpec((1, _NE), lambda i: (0, 0)),
        ],
        out_specs=[
            pl.BlockSpec((_BT, _NE), lambda i: (i, 0)),
            pl.BlockSpec((_BT, _K), lambda i: (i, 0)),
        ],
        out_shape=[
            jax.ShapeDtypeStruct((_NT, _NE), jnp.float32),
            jax.ShapeDtypeStruct((_NT, _K), jnp.int32),
        ],
    )(inputs, inputs, whl, bb)
    return (p, idx)


# speedup vs baseline: 1.0728x; 1.0167x over previous
import jax
import jax.numpy as jnp
from jax.experimental import pallas as pl
from jax.experimental.pallas import tpu as pltpu

_EMBED = 4096
_NE = 64
_K = 8
_NT = 32768
_BT = 1024


def _body(x1_ref, x2_ref, w_ref, b_ref, p_ref, idx_ref, wt_ref):
    @pl.when(pl.program_id(0) == 0)
    def _prep():
        wt_ref[...] = w_ref[...].astype(jnp.bfloat16).T

    xh1 = x1_ref[...].astype(jnp.bfloat16)
    xh2 = x2_ref[...].astype(jnp.bfloat16)
    acc = jnp.dot(xh1, wt_ref[: _EMBED // 2], preferred_element_type=jnp.float32)
    acc += jnp.dot(xh2, wt_ref[_EMBED // 2 :], preferred_element_type=jnp.float32)
    logits = acc + b_ref[...]

    lt = logits.T
    m = jnp.max(lt, axis=0, keepdims=True)
    e = jnp.exp(lt - m)
    s = jnp.sum(e, axis=0, keepdims=True)
    p_ref[...] = (e / s).T

    vals = lt
    iota = jax.lax.broadcasted_iota(jnp.int32, (_NE, lt.shape[1]), 0)
    rows = []
    for _ in range(_K):
        mx = jnp.max(vals, axis=0, keepdims=True)
        amin = jnp.min(jnp.where(vals >= mx, iota, _NE), axis=0, keepdims=True)
        rows.append(amin)
        vals = jnp.where(iota == amin, -jnp.inf, vals)
    idx_ref[...] = jnp.concatenate(rows, axis=0).T


@jax.jit
def kernel(inputs, W, b):
    bb = b.reshape(1, _NE)
    grid = (_NT // _BT,)
    p, idx = pl.pallas_call(
        _body,
        grid=grid,
        in_specs=[
            pl.BlockSpec((_BT, _EMBED // 2), lambda i: (i, 0)),
            pl.BlockSpec((_BT, _EMBED // 2), lambda i: (i, 1)),
            pl.BlockSpec((_NE, _EMBED), lambda i: (0, 0)),
            pl.BlockSpec((1, _NE), lambda i: (0, 0)),
        ],
        out_specs=[
            pl.BlockSpec((_BT, _NE), lambda i: (i, 0)),
            pl.BlockSpec((_BT, _K), lambda i: (i, 0)),
        ],
        out_shape=[
            jax.ShapeDtypeStruct((_NT, _NE), jnp.float32),
            jax.ShapeDtypeStruct((_NT, _K), jnp.int32),
        ],
        scratch_shapes=[pltpu.VMEM((_EMBED, _NE), jnp.bfloat16)],
    )(inputs, inputs, W, bb)
    return (p, idx)
